# baseline (speedup 1.0000x reference)
import jax
import jax.numpy as jnp
from jax.experimental import pallas as pl
from jax.experimental.pallas import tpu as pltpu

E = 8
K = 2
D_IN = 1024
D_H = 256
D_OUT = 1024
T = 2048

EPAD = 128
NEG = -1e30


def _moe_kernel(x_ref, wg_ref, bg_ref, w1_ref, b1_ref, w2_ref, b2_ref,
                final_ref, masked_ref, gates_ref, idx_ref,
                a1_s, a2_s, w0_s, w1s_s, xb_s, h_s):
    s = pl.program_id(0)

    @pl.when(s == 0)
    def _gating():
        xb_s[...] = x_ref[...].astype(jnp.bfloat16)
        wgp = jnp.pad(wg_ref[...], ((0, 0), (0, EPAD - E)))
        bgp = jnp.pad(bg_ref[...], ((0, 0), (0, EPAD - E)),
                      constant_values=NEG)
        logits = jnp.dot(x_ref[...], wgp,
                         preferred_element_type=jnp.float32) + bgp
        m = jnp.max(logits, axis=1, keepdims=True)
        p = jnp.exp(logits - m)
        g = p / jnp.sum(p, axis=1, keepdims=True)
        gates_ref[...] = g[:, :E]
        lane = jax.lax.broadcasted_iota(jnp.int32, g.shape, 1)
        m1 = jnp.max(g, axis=1, keepdims=True)
        a1 = jnp.min(jnp.where(g == m1, lane, EPAD), axis=1, keepdims=True)
        g2 = jnp.where(lane == a1, -1.0, g)
        m2 = jnp.max(g2, axis=1, keepdims=True)
        a2 = jnp.min(jnp.where(g2 == m2, lane, EPAD), axis=1, keepdims=True)
        ssum = m1 + m2
        a1_s[...] = a1
        a2_s[...] = a2
        w0_s[...] = m1 / ssum
        w1s_s[...] = m2 / ssum
        lane2 = jax.lax.broadcasted_iota(jnp.int32, (T, K), 1)
        idx_ref[...] = jnp.where(lane2 == 0, a1, a2)

    @pl.when(s < E)
    def _produce():
        h_s[pl.ds((s % 2) * T, T), :] = jnp.dot(
            xb_s[...], w1_ref[0].astype(jnp.bfloat16),
            preferred_element_type=jnp.float32)

    @pl.when(s > 0)
    def _consume():
        ec = s - 1
        h = jnp.maximum(h_s[pl.ds((ec % 2) * T, T), :] + b1_ref[0], 0.0)

        sel1 = a1_s[...] == ec
        sel2 = a2_s[...] == ec
        colm = (sel1 | sel2).astype(jnp.float32)
        colw = (jnp.where(sel1, w0_s[...], 0.0)
                + jnp.where(sel2, w1s_s[...], 0.0))

        hm = (h * colm).astype(jnp.bfloat16)
        mout = (jnp.dot(hm, w2_ref[0].astype(jnp.bfloat16),
                        preferred_element_type=jnp.float32)
                + colm * b2_ref[0])
        masked_ref[0] = mout

        @pl.when(s == 1)
        def _init():
            final_ref[...] = colw * mout

        @pl.when(s > 1)
        def _acc():
            final_ref[...] += colw * mout


@jax.jit
def kernel(x, Wg, bg, W1, b1, W2, b2):
    b1r = b1[:, None, :]
    b2r = b2[:, None, :]

    out_shapes = (
        jax.ShapeDtypeStruct((T, D_OUT), jnp.float32),
        jax.ShapeDtypeStruct((E, T, D_OUT), jnp.float32),
        jax.ShapeDtypeStruct((T, E), jnp.float32),
        jax.ShapeDtypeStruct((T, K), jnp.int32),
    )

    def prev(s):
        return jnp.maximum(s, 1) - 1

    return pl.pallas_call(
        _moe_kernel,
        grid=(E + 1,),
        in_specs=[
            pl.BlockSpec((T, D_IN), lambda s: (0, 0)),
            pl.BlockSpec((D_IN, E), lambda s: (0, 0)),
            pl.BlockSpec((1, E), lambda s: (0, 0)),
            pl.BlockSpec((1, D_IN, D_H),
                         lambda s: (jnp.minimum(s, E - 1), 0, 0)),
            pl.BlockSpec((1, 1, D_H), lambda s: (prev(s), 0, 0)),
            pl.BlockSpec((1, D_H, D_OUT), lambda s: (prev(s), 0, 0)),
            pl.BlockSpec((1, 1, D_OUT), lambda s: (prev(s), 0, 0)),
        ],
        out_specs=(
            pl.BlockSpec((T, D_OUT), lambda s: (0, 0)),
            pl.BlockSpec((1, T, D_OUT), lambda s: (prev(s), 0, 0)),
            pl.BlockSpec((T, E), lambda s: (0, 0)),
            pl.BlockSpec((T, K), lambda s: (0, 0)),
        ),
        out_shape=out_shapes,
        scratch_shapes=[
            pltpu.VMEM((T, 1), jnp.int32),
            pltpu.VMEM((T, 1), jnp.int32),
            pltpu.VMEM((T, 1), jnp.float32),
            pltpu.VMEM((T, 1), jnp.float32),
            pltpu.VMEM((T, D_IN), jnp.bfloat16),
            pltpu.VMEM((2 * T, D_H), jnp.float32),
        ],
        compiler_params=pltpu.CompilerParams(
            dimension_semantics=("arbitrary",),
            vmem_limit_bytes=100 * 1024 * 1024,
        ),
    )(x, Wg, bg[None, :], W1, b1r, W2, b2r)
